# Initial kernel scaffold; baseline (speedup 1.0000x reference)
import jax
import jax.numpy as jnp
from jax.experimental import pallas as pl


def kernel(input, edge_index, adj_values, weight, bias):
    raise NotImplementedError("write your pallas kernel here")


# speedup vs baseline: 4.4619x; 4.4619x over previous
import functools

import jax
import jax.numpy as jnp
from jax import lax
from jax.experimental import pallas as pl
from jax.experimental.pallas import tpu as pltpu
from jax.experimental.pallas import tpu_sc as plsc

NC = 2
NS = 16
NW = NC * NS
L = 16


def _matmul(x, w):
    n, din = x.shape
    dout = w.shape[1]
    bm = 1000
    assert n % bm == 0

    def body(x_ref, w_ref, o_ref):
        o_ref[...] = jnp.dot(x_ref[...], w_ref[...],
                             preferred_element_type=jnp.float32)

    return pl.pallas_call(
        body,
        out_shape=jax.ShapeDtypeStruct((n, dout), jnp.float32),
        grid=(n // bm,),
        in_specs=[
            pl.BlockSpec((bm, din), lambda i: (i, 0)),
            pl.BlockSpec((din, dout), lambda i: (0, 0)),
        ],
        out_specs=pl.BlockSpec((bm, dout), lambda i: (i, 0)),
    )(x, w)


def _combine(partials, bias2d, n):
    d = partials.shape[2]
    bm = 1000
    assert n % bm == 0

    def body(p_ref, b_ref, o_ref):
        o_ref[...] = p_ref[0] + p_ref[1] + b_ref[...]

    return pl.pallas_call(
        body,
        out_shape=jax.ShapeDtypeStruct((n, d), jnp.float32),
        grid=(n // bm,),
        in_specs=[
            pl.BlockSpec((2, bm, d), lambda i: (0, i, 0)),
            pl.BlockSpec((1, d), lambda i: (0, 0)),
        ],
        out_specs=pl.BlockSpec((bm, d), lambda i: (i, 0)),
    )(partials, bias2d)


def _sc_scatter(support, dst, src, adj):
    n, d = support.shape
    e = dst.shape[0]
    assert d == 8 * L
    ep = e // NW
    assert ep * NW == e
    C = 80
    nchunk = ep // C
    assert nchunk * C == ep
    ZR = 160
    np_ = ((n + ZR * NS - 1) // (ZR * NS)) * (ZR * NS)
    rows_per_tile = np_ // NS
    nz = rows_per_tile // ZR
    assert nz * ZR == rows_per_tile

    mesh = plsc.VectorSubcoreMesh(core_axis_name="c", subcore_axis_name="s",
                                  num_cores=NC, num_subcores=NS)

    @functools.partial(
        pl.kernel,
        out_type=jax.ShapeDtypeStruct((NC, np_, d), jnp.float32),
        mesh=mesh,
        scratch_types=[
            pltpu.VMEM_SHARED((np_, d), jnp.float32),
            pltpu.VMEM((C,), jnp.int32),
            pltpu.VMEM((C,), jnp.int32),
            pltpu.VMEM((C,), jnp.float32),
            pltpu.VMEM((C, d), jnp.float32),
            pltpu.VMEM((ZR, d), jnp.float32),
            pltpu.SemaphoreType.DMA,
        ],
    )
    def scatter_kernel(sup_hbm, dst_hbm, src_hbm, adj_hbm, out_hbm,
                       acc_sh, idx_v, dst_v, adj_v, rows_v, zero_v, sem):
        cid = lax.axis_index("c")
        sid = lax.axis_index("s")
        wid = sid * NC + cid

        zero = jnp.zeros((L,), jnp.float32)

        def zrow(i, carry):
            for f in range(d // L):
                zero_v[i, pl.ds(f * L, L)] = zero
            return carry

        lax.fori_loop(0, ZR, zrow, 0)
        for j in range(nz):
            pltpu.sync_copy(zero_v,
                            acc_sh.at[pl.ds(sid * rows_per_tile + j * ZR, ZR)])
        plsc.subcore_barrier()

        base0 = wid * ep

        def chunk_body(ci, carry):
            b = base0 + ci * C
            pltpu.sync_copy(src_hbm.at[pl.ds(b, C)], idx_v)
            pltpu.sync_copy(dst_hbm.at[pl.ds(b, C)], dst_v)
            pltpu.sync_copy(adj_hbm.at[pl.ds(b, C)], adj_v)
            pltpu.async_copy(sup_hbm.at[idx_v], rows_v, sem).wait()

            def group(gi, icarry):
                avec = adj_v[pl.ds(gi * L, L)]
                for lane in range(L):
                    ab = avec[lane]
                    ei = gi * L + lane
                    for f in range(d // L):
                        sl = pl.ds(f * L, L)
                        rows_v[ei, sl] = rows_v[ei, sl] * ab
                return icarry

            lax.fori_loop(0, C // L, group, 0)
            pltpu.sync_copy(rows_v, acc_sh.at[dst_v], add=True)
            return carry

        lax.fori_loop(0, nchunk, chunk_body, 0)
        plsc.subcore_barrier()
        pltpu.sync_copy(
            acc_sh.at[pl.ds(sid * rows_per_tile, rows_per_tile)],
            out_hbm.at[cid, pl.ds(sid * rows_per_tile, rows_per_tile)])

    return scatter_kernel(support, dst, src, adj)


def kernel(input, edge_index, adj_values, weight, bias):
    support = _matmul(input, weight)
    partials = _sc_scatter(support, edge_index[0], edge_index[1], adj_values)
    return _combine(partials, bias.reshape(1, -1), input.shape[0])


# speedup vs baseline: 8.4655x; 1.8973x over previous
import functools

import jax
import jax.numpy as jnp
from jax import lax
from jax.experimental import pallas as pl
from jax.experimental.pallas import tpu as pltpu
from jax.experimental.pallas import tpu_sc as plsc

NC = 2
NS = 16
NW = NC * NS
L = 16


def _matmul(x, w):
    n, din = x.shape
    dout = w.shape[1]
    bm = 1000
    assert n % bm == 0

    def body(x_ref, w_ref, o_ref):
        o_ref[...] = jnp.dot(x_ref[...], w_ref[...],
                             preferred_element_type=jnp.float32)

    return pl.pallas_call(
        body,
        out_shape=jax.ShapeDtypeStruct((n, dout), jnp.float32),
        grid=(n // bm,),
        in_specs=[
            pl.BlockSpec((bm, din), lambda i: (i, 0)),
            pl.BlockSpec((din, dout), lambda i: (0, 0)),
        ],
        out_specs=pl.BlockSpec((bm, dout), lambda i: (i, 0)),
    )(x, w)


def _combine(partials, bias2d, n):
    d = partials.shape[2]
    bm = 1000
    assert n % bm == 0

    def body(p_ref, b_ref, o_ref):
        o_ref[...] = p_ref[0] + p_ref[1] + b_ref[...]

    return pl.pallas_call(
        body,
        out_shape=jax.ShapeDtypeStruct((n, d), jnp.float32),
        grid=(n // bm,),
        in_specs=[
            pl.BlockSpec((2, bm, d), lambda i: (0, i, 0)),
            pl.BlockSpec((1, d), lambda i: (0, 0)),
        ],
        out_specs=pl.BlockSpec((bm, d), lambda i: (i, 0)),
    )(partials, bias2d)


def _sc_scatter(support, dst, src, adj):
    n, d = support.shape
    e = dst.shape[0]
    assert d == 8 * L
    ep = e // NW
    assert ep * NW == e
    C = 80
    NB = 4
    nchunk = ep // C
    assert nchunk * C == ep
    ngroup = (nchunk - 1) // NB
    assert ngroup * NB + 1 == nchunk
    np_ = ((n + C * NS - 1) // (C * NS)) * (C * NS)
    rows_per_tile = np_ // NS
    nz = rows_per_tile // C
    assert nz * C == rows_per_tile

    src3 = src.reshape(NW, nchunk, 1, C)
    dst3 = dst.reshape(NW, nchunk, 1, C)
    meta = jnp.concatenate([src3, dst3], axis=2)
    adj3 = adj.reshape(NW, nchunk, 1, C)

    mesh = plsc.VectorSubcoreMesh(core_axis_name="c", subcore_axis_name="s",
                                  num_cores=NC, num_subcores=NS)

    @functools.partial(
        pl.kernel,
        out_type=jax.ShapeDtypeStruct((NC, np_, d), jnp.float32),
        mesh=mesh,
        scratch_types=[
            pltpu.VMEM_SHARED((np_, d), jnp.float32),
            pltpu.VMEM((2, NB, 2, C), jnp.int32),
            pltpu.VMEM((2, NB, 1, C), jnp.float32),
            pltpu.VMEM((NB, C, d), jnp.float32),
            pltpu.SemaphoreType.DMA,
            pltpu.SemaphoreType.DMA((NB,)),
            pltpu.SemaphoreType.DMA((NB,)),
        ],
    )
    def scatter_kernel(sup_hbm, meta_hbm, adj_hbm, out_hbm,
                       acc_sh, meta_v, adj_v, rows_v, isem, gsem, ssem):
        cid = lax.axis_index("c")
        sid = lax.axis_index("s")
        wid = sid * NC + cid

        zero = jnp.zeros((L,), jnp.float32)

        def zrow(i, carry):
            for f in range(d // L):
                rows_v[0, i, pl.ds(f * L, L)] = zero
            return carry

        lax.fori_loop(0, C, zrow, 0)
        for j in range(nz):
            pltpu.sync_copy(rows_v.at[0],
                            acc_sh.at[pl.ds(sid * rows_per_tile + j * C, C)])
        plsc.subcore_barrier()

        pltpu.sync_copy(meta_hbm.at[wid, pl.ds(0, NB)], meta_v.at[0])
        pltpu.sync_copy(adj_hbm.at[wid, pl.ds(0, NB)], adj_v.at[0])
        pltpu.async_copy(meta_hbm.at[wid, pl.ds(NB, NB)], meta_v.at[1], isem)
        pltpu.async_copy(adj_hbm.at[wid, pl.ds(NB, NB)], adj_v.at[1], isem)
        for b in range(NB):
            pltpu.async_copy(sup_hbm.at[meta_v.at[0, b, 0]], rows_v.at[b],
                             gsem.at[b])

        def scale_chunk(rb, ms, mb):
            def sgroup(gi, ic):
                avec = adj_v[ms, mb, 0, pl.ds(gi * L, L)]
                for lane in range(L):
                    ab = avec[lane]
                    ei = gi * L + lane
                    for f in range(d // L):
                        sl = pl.ds(f * L, L)
                        rb[ei, sl] = rb[ei, sl] * ab
                return ic

            lax.fori_loop(0, C // L, sgroup, 0)

        def group_iter(G, carry):
            s = lax.rem(G, 2)
            for b in range(NB):
                rb = rows_v.at[b]
                pltpu.make_async_copy(sup_hbm.at[meta_v.at[s, b, 0]], rb,
                                      gsem.at[b]).wait()
                scale_chunk(rb, s, b)
                pltpu.async_copy(rb, acc_sh.at[meta_v.at[s, b, 1]],
                                 ssem.at[b], add=True)

            @pl.when(G < ngroup - 1)
            def _():
                sn = 1 - s
                for b in range(NB):
                    pltpu.make_async_copy(rows_v.at[b],
                                          acc_sh.at[meta_v.at[s, b, 1]],
                                          ssem.at[b]).wait()
                pltpu.make_async_copy(meta_hbm.at[wid, pl.ds(0, NB)],
                                      meta_v.at[sn], isem).wait()
                pltpu.make_async_copy(adj_hbm.at[wid, pl.ds(0, NB)],
                                      adj_v.at[sn], isem).wait()

                @pl.when(G < ngroup - 2)
                def _():
                    pltpu.async_copy(
                        meta_hbm.at[wid, pl.ds((G + 2) * NB, NB)],
                        meta_v.at[s], isem)
                    pltpu.async_copy(
                        adj_hbm.at[wid, pl.ds((G + 2) * NB, NB)],
                        adj_v.at[s], isem)

                for b in range(NB):
                    pltpu.async_copy(sup_hbm.at[meta_v.at[sn, b, 0]],
                                     rows_v.at[b], gsem.at[b])
            return carry

        lax.fori_loop(0, ngroup, group_iter, 0)

        sl_ = (ngroup - 1) % 2
        for b in range(NB):
            pltpu.make_async_copy(rows_v.at[b],
                                  acc_sh.at[meta_v.at[sl_, b, 1]],
                                  ssem.at[b]).wait()
        pltpu.sync_copy(meta_hbm.at[wid, pl.ds(nchunk - 1, 1)],
                        meta_v.at[0, pl.ds(0, 1)])
        pltpu.sync_copy(adj_hbm.at[wid, pl.ds(nchunk - 1, 1)],
                        adj_v.at[0, pl.ds(0, 1)])
        rb = rows_v.at[0]
        pltpu.async_copy(sup_hbm.at[meta_v.at[0, 0, 0]], rb,
                         gsem.at[0]).wait()
        scale_chunk(rb, 0, 0)
        pltpu.sync_copy(rb, acc_sh.at[meta_v.at[0, 0, 1]], add=True)

        plsc.subcore_barrier()
        pltpu.sync_copy(
            acc_sh.at[pl.ds(sid * rows_per_tile, rows_per_tile)],
            out_hbm.at[cid, pl.ds(sid * rows_per_tile, rows_per_tile)])

    return scatter_kernel(support, meta, adj3)


def kernel(input, edge_index, adj_values, weight, bias):
    support = _matmul(input, weight)
    partials = _sc_scatter(support, edge_index[0], edge_index[1], adj_values)
    return _combine(partials, bias.reshape(1, -1), input.shape[0])
